# baseline (speedup 1.0000x reference)
import functools

import jax
import jax.numpy as jnp
from jax import lax
from jax.experimental import pallas as pl
from jax.experimental.pallas import tpu as pltpu
from jax.experimental.pallas import tpu_sc as plsc

B, D, C, K = 4096, 128, 100000, 50000
CP = 102400
R = CP // 128
KP = 50176
NW = 32
CHUNK = CP // NW
TCH = B // NW
CW = 3328
CBLK = 1024
KBLK = 512
DUMPW = KP
SUBN = 53376
DUMPI = SUBN - 64
NQ = 5


def _nrm(x, eps):
    n = jnp.linalg.norm(x, ord=2, axis=-1, keepdims=True)
    return x / jnp.maximum(n, eps)


def _scores_body(tn_ref, wn_ref, te_ref, out_ref):
    i = pl.program_id(0)
    cos = lax.dot_general(tn_ref[...], wn_ref[...], (((1,), (1,)), ((), ())),
                          preferred_element_type=jnp.float32)
    ri = lax.broadcasted_iota(jnp.int32, (16, CBLK), 0)
    ci = lax.broadcasted_iota(jnp.int32, (16, CBLK), 1) + i * CBLK
    q = ri // 3
    kind = ri - q * 3
    chunk = (ci >> (4 * q)) & 15
    val = jnp.where(kind == 0, 1, jnp.where(kind == 1, chunk, chunk * chunk))
    val = jnp.where(ri >= 3 * NQ, 0, val).astype(jnp.float32)
    e2 = lax.dot_general(te_ref[...], val, (((1,), (0,)), ((), ())),
                         preferred_element_type=jnp.float32)
    out_ref[...] = jnp.where(jnp.max(e2, axis=0) == 0.0, 2.0,
                             jnp.max(cos, axis=0))


def _scores(tn, wn, tenc):
    return pl.pallas_call(
        _scores_body,
        grid=(98,),
        in_specs=[pl.BlockSpec((B, D), lambda i: (0, 0)),
                  pl.BlockSpec((CBLK, D), lambda i: (i, 0)),
                  pl.BlockSpec((B, 16), lambda i: (0, 0))],
        out_specs=pl.BlockSpec((CBLK,), lambda i: (i,)),
        out_shape=jax.ShapeDtypeStruct((CP,), jnp.float32),
    )(tn, wn, tenc)


def _target_enc(target):
    cols = []
    for q in range(NQ):
        t = ((target >> (4 * q)) & 15).astype(jnp.float32)
        cols += [-(t * t), 2.0 * t, -jnp.ones_like(t)]
    cols.append(jnp.zeros_like(cols[0]))
    return jnp.stack(cols, axis=1)


def _select_body(s_ref, sel_ref, pos_ref):
    s = s_ref[...]
    r = lax.broadcasted_iota(jnp.int32, (R, 128), 0)
    c = lax.broadcasted_iota(jnp.int32, (R, 128), 1)
    gi = r * 128 + c
    s = jnp.where(gi < C, s, -jnp.inf)
    u = lax.bitcast_convert_type(s, jnp.uint32)
    flip = jnp.where(u >= jnp.uint32(0x80000000),
                     jnp.uint32(0xFFFFFFFF), jnp.uint32(0x80000000))
    ku = u ^ flip

    def bit_step(it, t):
        cand = t | (jnp.uint32(0x80000000) >> it)
        cnt = jnp.sum((ku >= cand).astype(jnp.int32))
        return jnp.where(cnt >= K, cand, t)

    t = lax.fori_loop(0, 32, bit_step, jnp.uint32(0))

    gtb = ku > t
    eqb = ku == t
    gt = gtb.astype(jnp.float32)
    eq = eqb.astype(jnp.float32)
    need_eq = jnp.float32(K) - jnp.sum(gt)

    iu = lax.broadcasted_iota(jnp.int32, (128, 128), 0)
    ju = lax.broadcasted_iota(jnp.int32, (128, 128), 1)
    upper = (iu <= ju).astype(jnp.float32)
    ir = lax.broadcasted_iota(jnp.int32, (R, R), 0)
    jr = lax.broadcasted_iota(jnp.int32, (R, R), 1)
    lower = (ir > jr).astype(jnp.float32)

    def prefix(m):
        inrow = lax.dot_general(m, upper, (((1,), (0,)), ((), ())),
                                preferred_element_type=jnp.float32)
        rowtot = jnp.broadcast_to(jnp.sum(m, axis=1, keepdims=True), (R, 128))
        off = lax.dot_general(lower, rowtot, (((1,), (0,)), ((), ())),
                              preferred_element_type=jnp.float32)
        return off + inrow - m

    pgt = prefix(gt)
    peq = prefix(eq)
    seli = gtb | (eqb & (peq < need_eq))
    pos = pgt + jnp.minimum(peq, need_eq)
    sel_ref[...] = seli.astype(jnp.int32)
    pos_ref[...] = pos.astype(jnp.int32)


def _select(scores2):
    return pl.pallas_call(
        _select_body,
        out_shape=[jax.ShapeDtypeStruct((R, 128), jnp.int32),
                   jax.ShapeDtypeStruct((R, 128), jnp.int32)],
    )(scores2.reshape(R, 128))


def _stage_body(sel_hbm, pos_hbm, tgt_hbm, sub_hbm, ntgt_hbm,
                sel_v, pos_v, pix_v, gvl_v, tg_v, ntg_v, sem, sem2):
    wid = lax.axis_index("s") * 2 + lax.axis_index("c")
    base = wid * CHUNK
    pltpu.sync_copy(sel_hbm.at[pl.ds(base, CHUNK)], sel_v)
    pltpu.sync_copy(pos_hbm.at[pl.ds(base, CHUNK)], pos_v)
    lane = lax.iota(jnp.int32, 16)

    def bf(j, carry):
        r = j // 8
        cb = (j - r * 8) * 16
        s16 = sel_v[pl.ds(j * 16, 16)]
        p16 = pos_v[pl.ds(j * 16, 16)]
        g16 = base + j * 16 + lane
        pix_v[r, pl.ds(cb, 16)] = jnp.where(s16 > 0, p16, DUMPI + (g16 & 63))
        gvl_v[r, pl.ds(cb, 16)] = g16
        return carry

    lax.fori_loop(0, CHUNK // 16, bf, 0)

    def sf(rr, carry):
        pltpu.async_copy(gvl_v.at[rr], sub_hbm.at[pix_v.at[rr]], sem).wait()
        return carry

    lax.fori_loop(0, CHUNK // 128, sf, 0)

    tb = wid * TCH
    pltpu.sync_copy(tgt_hbm.at[pl.ds(tb, TCH)], tg_v)
    pltpu.async_copy(pos_hbm.at[tg_v], ntg_v, sem2).wait()
    pltpu.sync_copy(ntg_v, ntgt_hbm.at[pl.ds(tb, TCH)])


KSL = KP // NW
SUBR = 112


def _gather_body(sub_hbm, w_hbm, wsub_hbm, idx_v, idx2_v, rows_v, sem):
    wid = lax.axis_index("s") * 2 + lax.axis_index("c")
    base = wid * KSL
    pltpu.sync_copy(sub_hbm.at[pl.ds(base, KSL)], idx_v)

    def clf(u2, carry):
        v = idx_v[pl.ds(u2 * 16, 16)]
        r = u2 // 7
        cb = (u2 - r * 7) * 16
        idx2_v[r, pl.ds(cb, 16)] = jnp.where((v >= 0) & (v < C), v, 0)
        return carry

    lax.fori_loop(0, KSL // 16, clf, 0)

    def df(k2, carry):
        pltpu.async_copy(w_hbm.at[idx2_v.at[k2]], rows_v, sem).wait()
        pltpu.sync_copy(rows_v, wsub_hbm.at[pl.ds(base + k2 * SUBR, SUBR)])
        return carry

    lax.fori_loop(0, KSL // SUBR, df, 0)


@functools.lru_cache(maxsize=None)
def _sc_kernels():
    mesh = plsc.VectorSubcoreMesh(core_axis_name="c", subcore_axis_name="s",
                                  num_cores=2, num_subcores=16)
    stage = pl.kernel(
        _stage_body,
        out_type=[jax.ShapeDtypeStruct((SUBN,), jnp.int32),
                  jax.ShapeDtypeStruct((B,), jnp.int32)],
        mesh=mesh,
        scratch_types=[pltpu.VMEM((CHUNK,), jnp.int32),
                       pltpu.VMEM((CHUNK,), jnp.int32),
                       pltpu.VMEM((CHUNK // 128, 128), jnp.int32),
                       pltpu.VMEM((CHUNK // 128, 128), jnp.int32),
                       pltpu.VMEM((TCH,), jnp.int32),
                       pltpu.VMEM((TCH,), jnp.int32),
                       pltpu.SemaphoreType.DMA,
                       pltpu.SemaphoreType.DMA],
    )
    gather = pl.kernel(
        _gather_body,
        out_type=jax.ShapeDtypeStruct((KP + 8, D), jnp.float32),
        mesh=mesh,
        scratch_types=[pltpu.VMEM((KSL,), jnp.int32),
                       pltpu.VMEM((KSL // SUBR, SUBR), jnp.int32),
                       pltpu.VMEM((SUBR, D), jnp.float32),
                       pltpu.SemaphoreType.DMA],
    )
    return stage, gather


def _logits_body(t2_ref, ws_ref, out_ref):
    w = ws_ref[...]
    n = jnp.sqrt(jnp.sum(w * w, axis=1, keepdims=True))
    w2 = w / jnp.maximum(n, 1e-12)
    a = t2_ref[...].astype(jnp.bfloat16)
    b2 = w2.astype(jnp.bfloat16)
    out_ref[...] = lax.dot_general(a, b2, (((1,), (1,)), ((), ())),
                                   preferred_element_type=jnp.float32)


def _logits(t2, wsub):
    return pl.pallas_call(
        _logits_body,
        grid=(KP // KBLK,),
        in_specs=[pl.BlockSpec((B, D), lambda i: (0, 0)),
                  pl.BlockSpec((KBLK, D), lambda i: (i, 0))],
        out_specs=pl.BlockSpec((B, KBLK), lambda i: (0, i)),
        out_shape=jax.ShapeDtypeStruct((B, K), jnp.float32),
    )(t2, wsub)


def kernel(tensor, target, weight):
    eps1 = float(jnp.finfo(jnp.float32).eps)
    tn = _nrm(tensor, eps1)
    wn = _nrm(weight, eps1)
    t2 = _nrm(tensor, 1e-12)
    tenc = _target_enc(target)
    scores2 = _scores(tn, wn, tenc)
    sel, pos = _select(scores2)
    stage, gather = _sc_kernels()
    sub, ntgt = stage(sel.reshape(CP), pos.reshape(CP), target)
    wsub = gather(sub, weight)
    logits = _logits(t2, wsub)
    return (logits, ntgt)


# speedup vs baseline: 5.3476x; 5.3476x over previous
import functools

import jax
import jax.numpy as jnp
from jax import lax
from jax.experimental import pallas as pl
from jax.experimental.pallas import tpu as pltpu
from jax.experimental.pallas import tpu_sc as plsc

B, D, C, K = 4096, 128, 100000, 50000
CP = 102400
R = CP // 128
KP = 50176
NW = 32
CHUNK = CP // NW
TCH = B // NW
CW = 3328
CBLK = 1024
KBLK = 512
DUMPW = KP
DUMPI = 53376
SUBN = DUMPI + CP
NQ = 5


def _nrm(x, eps):
    n = jnp.linalg.norm(x, ord=2, axis=-1, keepdims=True)
    return x / jnp.maximum(n, eps)


def _scores_body(tn_ref, wn_ref, te_ref, out_ref):
    i = pl.program_id(0)
    cos = lax.dot_general(tn_ref[...], wn_ref[...], (((1,), (1,)), ((), ())),
                          preferred_element_type=jnp.float32)
    ri = lax.broadcasted_iota(jnp.int32, (16, CBLK), 0)
    ci = lax.broadcasted_iota(jnp.int32, (16, CBLK), 1) + i * CBLK
    q = ri // 3
    kind = ri - q * 3
    chunk = (ci >> (4 * q)) & 15
    val = jnp.where(kind == 0, 1, jnp.where(kind == 1, chunk, chunk * chunk))
    val = jnp.where(ri >= 3 * NQ, 0, val).astype(jnp.float32)
    e2 = lax.dot_general(te_ref[...], val, (((1,), (0,)), ((), ())),
                         preferred_element_type=jnp.float32)
    out_ref[...] = jnp.where(jnp.max(e2, axis=0) == 0.0, 2.0,
                             jnp.max(cos, axis=0))


def _scores(tn, wn, tenc):
    return pl.pallas_call(
        _scores_body,
        grid=(98,),
        in_specs=[pl.BlockSpec((B, D), lambda i: (0, 0)),
                  pl.BlockSpec((CBLK, D), lambda i: (i, 0)),
                  pl.BlockSpec((B, 16), lambda i: (0, 0))],
        out_specs=pl.BlockSpec((CBLK,), lambda i: (i,)),
        out_shape=jax.ShapeDtypeStruct((CP,), jnp.float32),
    )(tn, wn, tenc)


def _target_enc(target):
    cols = []
    for q in range(NQ):
        t = ((target >> (4 * q)) & 15).astype(jnp.float32)
        cols += [-(t * t), 2.0 * t, -jnp.ones_like(t)]
    cols.append(jnp.zeros_like(cols[0]))
    return jnp.stack(cols, axis=1)


def _select_body(s_ref, sel_ref, pos_ref):
    s = s_ref[...]
    r = lax.broadcasted_iota(jnp.int32, (R, 128), 0)
    c = lax.broadcasted_iota(jnp.int32, (R, 128), 1)
    gi = r * 128 + c
    s = jnp.where(gi < C, s, -jnp.inf)
    u = lax.bitcast_convert_type(s, jnp.uint32)
    flip = jnp.where(u >= jnp.uint32(0x80000000),
                     jnp.uint32(0xFFFFFFFF), jnp.uint32(0x80000000))
    ku = u ^ flip

    def bit_step(it, t):
        cand = t | (jnp.uint32(0x80000000) >> it)
        cnt = jnp.sum((ku >= cand).astype(jnp.int32))
        return jnp.where(cnt >= K, cand, t)

    t = lax.fori_loop(0, 32, bit_step, jnp.uint32(0))

    gtb = ku > t
    eqb = ku == t
    gt = gtb.astype(jnp.float32)
    eq = eqb.astype(jnp.float32)
    need_eq = jnp.float32(K) - jnp.sum(gt)

    iu = lax.broadcasted_iota(jnp.int32, (128, 128), 0)
    ju = lax.broadcasted_iota(jnp.int32, (128, 128), 1)
    upper = (iu <= ju).astype(jnp.float32)
    ir = lax.broadcasted_iota(jnp.int32, (R, R), 0)
    jr = lax.broadcasted_iota(jnp.int32, (R, R), 1)
    lower = (ir > jr).astype(jnp.float32)

    def prefix(m):
        inrow = lax.dot_general(m, upper, (((1,), (0,)), ((), ())),
                                preferred_element_type=jnp.float32)
        rowtot = jnp.broadcast_to(jnp.sum(m, axis=1, keepdims=True), (R, 128))
        off = lax.dot_general(lower, rowtot, (((1,), (0,)), ((), ())),
                              preferred_element_type=jnp.float32)
        return off + inrow - m

    pgt = prefix(gt)
    peq = prefix(eq)
    seli = gtb | (eqb & (peq < need_eq))
    pos = pgt + jnp.minimum(peq, need_eq)
    sel_ref[...] = seli.astype(jnp.int32)
    pos_ref[...] = pos.astype(jnp.int32)


def _select(scores2):
    return pl.pallas_call(
        _select_body,
        out_shape=[jax.ShapeDtypeStruct((R, 128), jnp.int32),
                   jax.ShapeDtypeStruct((R, 128), jnp.int32)],
    )(scores2.reshape(R, 128))


def _stage_body(sel_hbm, pos_hbm, tgt_hbm, sub_hbm, ntgt_hbm,
                sel_v, pos_v, pix_v, gvl_v, tg_v, ntg_v, sem, sem2):
    wid = lax.axis_index("s") * 2 + lax.axis_index("c")
    base = wid * CHUNK
    pltpu.sync_copy(sel_hbm.at[pl.ds(base, CHUNK)], sel_v)
    pltpu.sync_copy(pos_hbm.at[pl.ds(base, CHUNK)], pos_v)
    lane = lax.iota(jnp.int32, 16)

    def bf(j, carry):
        r = j // 8
        cb = (j - r * 8) * 16
        s16 = sel_v[pl.ds(j * 16, 16)]
        p16 = pos_v[pl.ds(j * 16, 16)]
        g16 = base + j * 16 + lane
        pix_v[r, pl.ds(cb, 16)] = jnp.where(s16 > 0, p16, DUMPI + g16)
        gvl_v[r, pl.ds(cb, 16)] = g16
        return carry

    lax.fori_loop(0, CHUNK // 16, bf, 0)

    def sf(rr, carry):
        pltpu.async_copy(gvl_v.at[rr], sub_hbm.at[pix_v.at[rr]], sem).wait()
        return carry

    lax.fori_loop(0, CHUNK // 128, sf, 0)

    tb = wid * TCH
    pltpu.sync_copy(tgt_hbm.at[pl.ds(tb, TCH)], tg_v)
    pltpu.async_copy(pos_hbm.at[tg_v], ntg_v, sem2).wait()
    pltpu.sync_copy(ntg_v, ntgt_hbm.at[pl.ds(tb, TCH)])


KSL = KP // NW
SUBR = 112


def _gather_body(sub_hbm, w_hbm, wsub_hbm, idx_v, idx2_v, rows_v, sem):
    wid = lax.axis_index("s") * 2 + lax.axis_index("c")
    base = wid * KSL
    pltpu.sync_copy(sub_hbm.at[pl.ds(base, KSL)], idx_v)

    def clf(u2, carry):
        v = idx_v[pl.ds(u2 * 16, 16)]
        r = u2 // 7
        cb = (u2 - r * 7) * 16
        idx2_v[r, pl.ds(cb, 16)] = jnp.where((v >= 0) & (v < C), v, 0)
        return carry

    lax.fori_loop(0, KSL // 16, clf, 0)

    def df(k2, carry):
        pltpu.async_copy(w_hbm.at[idx2_v.at[k2]], rows_v, sem).wait()
        pltpu.sync_copy(rows_v, wsub_hbm.at[pl.ds(base + k2 * SUBR, SUBR)])
        return carry

    lax.fori_loop(0, KSL // SUBR, df, 0)


@functools.lru_cache(maxsize=None)
def _sc_kernels():
    mesh = plsc.VectorSubcoreMesh(core_axis_name="c", subcore_axis_name="s",
                                  num_cores=2, num_subcores=16)
    stage = pl.kernel(
        _stage_body,
        out_type=[jax.ShapeDtypeStruct((SUBN,), jnp.int32),
                  jax.ShapeDtypeStruct((B,), jnp.int32)],
        mesh=mesh,
        scratch_types=[pltpu.VMEM((CHUNK,), jnp.int32),
                       pltpu.VMEM((CHUNK,), jnp.int32),
                       pltpu.VMEM((CHUNK // 128, 128), jnp.int32),
                       pltpu.VMEM((CHUNK // 128, 128), jnp.int32),
                       pltpu.VMEM((TCH,), jnp.int32),
                       pltpu.VMEM((TCH,), jnp.int32),
                       pltpu.SemaphoreType.DMA,
                       pltpu.SemaphoreType.DMA],
    )
    gather = pl.kernel(
        _gather_body,
        out_type=jax.ShapeDtypeStruct((KP + 8, D), jnp.float32),
        mesh=mesh,
        scratch_types=[pltpu.VMEM((KSL,), jnp.int32),
                       pltpu.VMEM((KSL // SUBR, SUBR), jnp.int32),
                       pltpu.VMEM((SUBR, D), jnp.float32),
                       pltpu.SemaphoreType.DMA],
    )
    return stage, gather


def _logits_body(t2_ref, ws_ref, out_ref):
    w = ws_ref[...]
    n = jnp.sqrt(jnp.sum(w * w, axis=1, keepdims=True))
    w2 = w / jnp.maximum(n, 1e-12)
    a = t2_ref[...].astype(jnp.bfloat16)
    b2 = w2.astype(jnp.bfloat16)
    out_ref[...] = lax.dot_general(a, b2, (((1,), (1,)), ((), ())),
                                   preferred_element_type=jnp.float32)


def _logits(t2, wsub):
    return pl.pallas_call(
        _logits_body,
        grid=(KP // KBLK,),
        in_specs=[pl.BlockSpec((B, D), lambda i: (0, 0)),
                  pl.BlockSpec((KBLK, D), lambda i: (i, 0))],
        out_specs=pl.BlockSpec((B, KBLK), lambda i: (0, i)),
        out_shape=jax.ShapeDtypeStruct((B, K), jnp.float32),
    )(t2, wsub)


def kernel(tensor, target, weight):
    eps1 = float(jnp.finfo(jnp.float32).eps)
    tn = _nrm(tensor, eps1)
    wn = _nrm(weight, eps1)
    t2 = _nrm(tensor, 1e-12)
    tenc = _target_enc(target)
    scores2 = _scores(tn, wn, tenc)
    sel, pos = _select(scores2)
    stage, gather = _sc_kernels()
    sub, ntgt = stage(sel.reshape(CP), pos.reshape(CP), target)
    wsub = gather(sub, weight)
    logits = _logits(t2, wsub)
    return (logits, ntgt)
